# baseline (speedup 1.0000x reference)
import functools

import jax
import jax.numpy as jnp
from jax import lax
from jax.experimental import pallas as pl
from jax.experimental.pallas import tpu as pltpu
from jax.experimental.pallas import tpu_sc as plsc

_NC = 2
_NS = 16
_NW = _NC * _NS
_L = 16

_CH_E = 2000
_CH_A = 1568
_EUNROLL = 25
_AUNROLL = 14

_NSWEEP = 8


def _sc_partials(px, py, pz, tx, ty, tz, abatch, pelx, pely, pelz, ebatch,
                 N, E, B):
    KE = -(-E // (_NW * _CH_E))
    KA = -(-N // (_NW * _CH_A))

    ESTRIDE = 3 * B + 1
    ASTRIDE = 2 * B + 1

    mesh = plsc.VectorSubcoreMesh(
        core_axis_name="c", subcore_axis_name="s",
        num_cores=_NC, num_subcores=_NS)

    @functools.partial(
        pl.kernel,
        out_type=(jax.ShapeDtypeStruct((_NW, 3 * B), jnp.float32),
                  jax.ShapeDtypeStruct((_NW, 2 * B), jnp.float32)),
        mesh=mesh,
        compiler_params=pltpu.CompilerParams(needs_layout_passes=False),
        scratch_types=[
            pltpu.VMEM((_L * ESTRIDE,), jnp.float32),
            pltpu.VMEM((_L * ASTRIDE,), jnp.float32),
            pltpu.VMEM((_CH_E,), jnp.float32),
            pltpu.VMEM((_CH_E,), jnp.float32),
            pltpu.VMEM((_CH_E,), jnp.float32),
            pltpu.VMEM((_CH_E,), jnp.int32),
            pltpu.VMEM((_CH_E,), jnp.float32),
            pltpu.VMEM((_CH_E,), jnp.float32),
            pltpu.VMEM((_CH_E,), jnp.float32),
            pltpu.VMEM((_CH_E,), jnp.int32),
            pltpu.VMEM((_CH_A,), jnp.float32),
            pltpu.VMEM((_CH_A,), jnp.float32),
            pltpu.VMEM((_CH_A,), jnp.float32),
            pltpu.VMEM((_CH_A,), jnp.float32),
            pltpu.VMEM((_CH_A,), jnp.float32),
            pltpu.VMEM((_CH_A,), jnp.float32),
            pltpu.VMEM((_CH_A,), jnp.int32),
            pltpu.VMEM((_CH_A,), jnp.float32),
            pltpu.VMEM((_CH_A,), jnp.float32),
            pltpu.VMEM((_CH_A,), jnp.float32),
            pltpu.VMEM((_CH_A,), jnp.float32),
            pltpu.VMEM((_CH_A,), jnp.float32),
            pltpu.VMEM((_CH_A,), jnp.float32),
            pltpu.VMEM((_CH_A,), jnp.int32),
            pltpu.VMEM((3 * B,), jnp.float32),
            pltpu.VMEM((2 * B,), jnp.float32),
            pltpu.SemaphoreType.DMA,
            pltpu.SemaphoreType.DMA,
            pltpu.SemaphoreType.DMA,
            pltpu.SemaphoreType.DMA,
        ],
    )
    def sck(px_h, py_h, pz_h, tx_h, ty_h, tz_h, abatch_h,
            pelx_h, pely_h, pelz_h, ebatch_h,
            oute_h, outa_h,
            acc_e, acc_a,
            ex0, ey0, ez0, ei0, ex1, ey1, ez1, ei1,
            apx0, apy0, apz0, atx0, aty0, atz0, aid0,
            apx1, apy1, apz1, atx1, aty1, atz1, aid1,
            red_e, red_a, esem0, esem1, asem0, asem1):
        wid = lax.axis_index("s") * _NC + lax.axis_index("c")
        iota = lax.iota(jnp.int32, _L)
        zf = jnp.zeros((_L,), jnp.float32)
        onesf = zf + 1.0
        lane_e = iota * ESTRIDE
        lane_a = iota * ASTRIDE
        ebufs = ((ex0, ey0, ez0, ei0), (ex1, ey1, ez1, ei1))
        abufs = ((apx0, apy0, apz0, atx0, aty0, atz0, aid0),
                 (apx1, apy1, apz1, atx1, aty1, atz1, aid1))
        esems = (esem0, esem1)
        asems = (asem0, asem1)
        ebase0 = wid * (KE * _CH_E)
        abase0 = wid * (KA * _CH_A)

        def ecps(k, bufs):
            base = ebase0 + k * _CH_E
            return tuple(
                (src, src.at[pl.ds(base, _CH_E)], dst) for src, dst in
                zip((pelx_h, pely_h, pelz_h, ebatch_h), bufs))

        def acps(k, bufs):
            base = abase0 + k * _CH_A
            return tuple(
                (src, src.at[pl.ds(base, _CH_A)], dst) for src, dst in
                zip((px_h, py_h, pz_h, tx_h, ty_h, tz_h, abatch_h), bufs))

        def fire(cps, sem):
            for _, sl, dst in cps:
                pltpu.async_copy(sl, dst, sem)

        def drain(cps, sem):
            for _, sl, dst in cps:
                pltpu.make_async_copy(sl, dst, sem).wait()

        fire(ecps(0, ebufs[0]), esems[0])
        fire(acps(0, abufs[0]), asems[0])

        def zero_e(i, carry):
            acc_e[pl.ds(i * _L, _L)] = zf
            return carry
        lax.fori_loop(0, _L * ESTRIDE // _L, zero_e, None)

        def zero_a(i, carry):
            acc_a[pl.ds(i * _L, _L)] = zf
            return carry
        lax.fori_loop(0, _L * ASTRIDE // _L, zero_a, None)

        for k in range(KE):
            cb = k % 2
            nb = (k + 1) % 2
            if k + 1 < KE:
                fire(ecps(k + 1, ebufs[nb]), esems[nb])
            drain(ecps(k, ebufs[cb]), esems[cb])
            xv, yv, zv, iv = ebufs[cb]

            def estep(j, c2, xv=xv, yv=yv, zv=zv, iv=iv):
                for u in range(_EUNROLL):
                    o = (j * _EUNROLL + u) * _L
                    sl = pl.ds(o, _L)
                    ids = iv[sl]
                    si = lane_e + ids
                    plsc.addupdate_scatter(acc_e, [si], xv[sl])
                    plsc.addupdate_scatter(acc_e, [si + B], yv[sl])
                    plsc.addupdate_scatter(acc_e, [si + 2 * B], zv[sl])
                return c2
            lax.fori_loop(0, _CH_E // (_L * _EUNROLL), estep, None)

        for k in range(KA):
            cb = k % 2
            nb = (k + 1) % 2
            if k + 1 < KA:
                fire(acps(k + 1, abufs[nb]), asems[nb])
            drain(acps(k, abufs[cb]), asems[cb])
            pxv, pyv, pzv, txv, tyv, tzv, iv = abufs[cb]
            base = abase0 + k * _CH_A

            def astep(j, c2, pxv=pxv, pyv=pyv, pzv=pzv, txv=txv, tyv=tyv,
                      tzv=tzv, iv=iv, base=base):
                for u in range(_AUNROLL):
                    o = (j * _AUNROLL + u) * _L
                    sl = pl.ds(o, _L)
                    ids = iv[sl]
                    sq = zf
                    for pv, tv in ((pxv, txv), (pyv, tyv), (pzv, tzv)):
                        d = jnp.abs(pv[sl] - tv[sl])
                        d = jnp.minimum(d, 1.0)
                        w = jnp.minimum(d, 1.0 - d)
                        sq = sq + w * w
                    si = lane_a + ids
                    plsc.addupdate_scatter(acc_a, [si], sq)
                    gmask = (base + o + iota) < N
                    plsc.addupdate_scatter(acc_a, [si + B], onesf, mask=gmask)
                return c2
            lax.fori_loop(0, _CH_A // (_L * _AUNROLL), astep, None)

        def rede(v, carry):
            o = v * _L
            t = [acc_e[pl.ds(l * ESTRIDE + o, _L)] for l in range(_L)]
            while len(t) > 1:
                t = [t[i] + t[i + 1] for i in range(0, len(t), 2)]
            red_e[pl.ds(o, _L)] = t[0]
            return carry
        lax.fori_loop(0, (3 * B) // _L, rede, None)

        def reda(v, carry):
            o = v * _L
            t = [acc_a[pl.ds(l * ASTRIDE + o, _L)] for l in range(_L)]
            while len(t) > 1:
                t = [t[i] + t[i + 1] for i in range(0, len(t), 2)]
            red_a[pl.ds(o, _L)] = t[0]
            return carry
        lax.fori_loop(0, (2 * B) // _L, reda, None)

        pltpu.sync_copy(red_e, oute_h.at[wid])
        pltpu.sync_copy(red_a, outa_h.at[wid])

    return sck(px, py, pz, tx, ty, tz, abatch, pelx, pely, pelz, ebatch)


def _combine_body(oute, outa, lat, noise, out):
    B = noise.shape[1]
    e = oute[...]
    S = [jnp.sum(e[:, i * B:(i + 1) * B], axis=0) for i in range(3)]
    a = outa[...]
    seg_sum = jnp.sum(a[:, 0:B], axis=0)
    seg_cnt = jnp.sum(a[:, B:2 * B], axis=0)
    err_x = seg_sum / jnp.maximum(seg_cnt, 1.0)
    mean_err_x = jnp.sum(err_x) / B

    m = [[lat[3 * i + j, :] * S[i] for j in range(3)] for i in range(3)]
    A = [[None] * 3 for _ in range(3)]
    for j in range(3):
        for k in range(j, 3):
            A[j][k] = m[0][j] * m[0][k] + m[1][j] * m[1][k] + m[2][j] * m[2][k]
            A[k][j] = A[j][k]
    one = jnp.ones((B,), jnp.float32)
    zero = jnp.zeros((B,), jnp.float32)
    V = [[one if i == j else zero for j in range(3)] for i in range(3)]

    for _ in range(_NSWEEP):
        for (p, q) in ((0, 1), (0, 2), (1, 2)):
            app, aqq, apq = A[p][p], A[q][q], A[p][q]
            nz = apq != 0.0
            apq_s = jnp.where(nz, apq, 1.0)
            tau = (aqq - app) / (2.0 * apq_s)
            sgn = jnp.where(tau >= 0.0, 1.0, -1.0)
            t = sgn / (jnp.abs(tau) + jnp.sqrt(1.0 + tau * tau))
            t = jnp.where(nz, t, 0.0)
            c = 1.0 / jnp.sqrt(1.0 + t * t)
            s = t * c
            r = 3 - p - q
            arp = c * A[r][p] - s * A[r][q]
            arq = s * A[r][p] + c * A[r][q]
            A[p][p] = app - t * apq
            A[q][q] = aqq + t * apq
            A[p][q] = zero
            A[q][p] = zero
            A[r][p] = arp
            A[p][r] = arp
            A[r][q] = arq
            A[q][r] = arq
            for i in range(3):
                vip = c * V[i][p] - s * V[i][q]
                viq = s * V[i][p] + c * V[i][q]
                V[i][p] = vip
                V[i][q] = viq

    rt = [jnp.sqrt(jnp.maximum(A[k][k], 0.0)) for k in range(3)]

    def sym(i, j):
        return (V[i][0] * rt[0] * V[j][0]
                + V[i][1] * rt[1] * V[j][1]
                + V[i][2] * rt[2] * V[j][2])

    comps = (sym(0, 0), sym(1, 1), sym(2, 2), sym(0, 1), sym(0, 2), sym(1, 2))
    tot = jnp.zeros((), jnp.float32)
    for ci in range(6):
        d = comps[ci] - noise[ci, :]
        tot = tot + jnp.sum(d * d)
    err_l = tot / (6.0 * B)
    out[0, 0] = mean_err_x + err_l


def _tc_combine(oute, outa, lat_t, noise_t, interpret=False):
    res = pl.pallas_call(
        _combine_body,
        out_shape=jax.ShapeDtypeStruct((1, 1), jnp.float32),
        out_specs=pl.BlockSpec(memory_space=pltpu.SMEM),
        interpret=interpret,
    )(oute, outa, lat_t, noise_t)
    return res[0, 0]


def kernel(pred_frac_eps_x, target_frac_eps_x, atom_batch, neighbor_direction,
           pred_edge_distance_score, lattice, batch_of_edge,
           symmetric_vector_noise):
    B = lattice.shape[0]
    N = pred_frac_eps_x.shape[0]
    E = neighbor_direction.shape[0]
    KE = -(-E // (_NW * _CH_E))
    KA = -(-N // (_NW * _CH_A))
    EP = _NW * KE * _CH_E
    AP = _NW * KA * _CH_A
    pred = pred_frac_eps_x.astype(jnp.float32)
    targ = target_frac_eps_x.astype(jnp.float32)
    nd = neighbor_direction.astype(jnp.float32)
    sc = pred_edge_distance_score.astype(jnp.float32)[:, 0]
    ab = jnp.pad(atom_batch.astype(jnp.int32), (0, AP - N))
    eb = jnp.pad(batch_of_edge.astype(jnp.int32), (0, EP - E))
    pelx, pely, pelz = (jnp.pad(nd[:, i] * sc, (0, EP - E)) for i in range(3))
    px, py, pz, tx, ty, tz = (jnp.pad(a, (0, AP - N)) for a in (
        pred[:, 0], pred[:, 1], pred[:, 2],
        targ[:, 0], targ[:, 1], targ[:, 2]))
    oute, outa = _sc_partials(px, py, pz, tx, ty, tz, ab,
                              pelx, pely, pelz, eb, N, E, B)
    lat_t = jnp.transpose(jnp.reshape(lattice.astype(jnp.float32), (B, 9)))
    noise_t = jnp.transpose(symmetric_vector_noise.astype(jnp.float32))
    return _tc_combine(oute, outa, lat_t, noise_t)


# speedup vs baseline: 1.0672x; 1.0672x over previous
import functools

import jax
import jax.numpy as jnp
from jax import lax
from jax.experimental import pallas as pl
from jax.experimental.pallas import tpu as pltpu
from jax.experimental.pallas import tpu_sc as plsc

_NC = 2
_NS = 16
_NW = _NC * _NS
_L = 16

_CH_E = 2000
_CH_A = 1568
_EUNROLL = 5
_AUNROLL = 7

_NSWEEP = 8


def _sc_mesh():
    return plsc.VectorSubcoreMesh(
        core_axis_name="c", subcore_axis_name="s",
        num_cores=_NC, num_subcores=_NS)


def _sc_atoms(px, py, pz, tx, ty, tz, abatch, N, B):
    KA = -(-N // (_NW * _CH_A))
    ASTRIDE = 2 * B + 1

    @functools.partial(
        pl.kernel,
        out_type=jax.ShapeDtypeStruct((_NW, 2 * B), jnp.float32),
        mesh=_sc_mesh(),
        compiler_params=pltpu.CompilerParams(needs_layout_passes=False),
        scratch_types=[
            pltpu.VMEM((_L * ASTRIDE,), jnp.float32),
            pltpu.VMEM((_CH_A,), jnp.float32),
            pltpu.VMEM((_CH_A,), jnp.float32),
            pltpu.VMEM((_CH_A,), jnp.float32),
            pltpu.VMEM((_CH_A,), jnp.float32),
            pltpu.VMEM((_CH_A,), jnp.float32),
            pltpu.VMEM((_CH_A,), jnp.float32),
            pltpu.VMEM((_CH_A,), jnp.int32),
            pltpu.VMEM((_CH_A,), jnp.float32),
            pltpu.VMEM((_CH_A,), jnp.float32),
            pltpu.VMEM((_CH_A,), jnp.float32),
            pltpu.VMEM((_CH_A,), jnp.float32),
            pltpu.VMEM((_CH_A,), jnp.float32),
            pltpu.VMEM((_CH_A,), jnp.float32),
            pltpu.VMEM((_CH_A,), jnp.int32),
            pltpu.VMEM((2 * B,), jnp.float32),
            pltpu.SemaphoreType.DMA,
            pltpu.SemaphoreType.DMA,
        ],
    )
    def sck(px_h, py_h, pz_h, tx_h, ty_h, tz_h, abatch_h, outa_h,
            acc_a,
            apx0, apy0, apz0, atx0, aty0, atz0, aid0,
            apx1, apy1, apz1, atx1, aty1, atz1, aid1,
            red_a, asem0, asem1):
        wid = lax.axis_index("s") * _NC + lax.axis_index("c")
        iota = lax.iota(jnp.int32, _L)
        zf = jnp.zeros((_L,), jnp.float32)
        onesf = zf + 1.0
        lane_a = iota * ASTRIDE
        abufs = ((apx0, apy0, apz0, atx0, aty0, atz0, aid0),
                 (apx1, apy1, apz1, atx1, aty1, atz1, aid1))
        asems = (asem0, asem1)
        abase0 = wid * (KA * _CH_A)

        def acps(k, bufs):
            base = abase0 + k * _CH_A
            return tuple(
                (src, src.at[pl.ds(base, _CH_A)], dst) for src, dst in
                zip((px_h, py_h, pz_h, tx_h, ty_h, tz_h, abatch_h), bufs))

        def fire(cps, sem):
            for _, sl, dst in cps:
                pltpu.async_copy(sl, dst, sem)

        def drain(cps, sem):
            for _, sl, dst in cps:
                pltpu.make_async_copy(sl, dst, sem).wait()

        fire(acps(0, abufs[0]), asems[0])

        def zero_a(i, carry):
            acc_a[pl.ds(i * _L, _L)] = zf
            return carry
        lax.fori_loop(0, _L * ASTRIDE // _L, zero_a, None)

        for k in range(KA):
            cb = k % 2
            nb = (k + 1) % 2
            if k + 1 < KA:
                fire(acps(k + 1, abufs[nb]), asems[nb])
            drain(acps(k, abufs[cb]), asems[cb])
            pxv, pyv, pzv, txv, tyv, tzv, iv = abufs[cb]
            base = abase0 + k * _CH_A

            def astep(j, c2, pxv=pxv, pyv=pyv, pzv=pzv, txv=txv, tyv=tyv,
                      tzv=tzv, iv=iv, base=base):
                for u in range(_AUNROLL):
                    o = (j * _AUNROLL + u) * _L
                    sl = pl.ds(o, _L)
                    ids = iv[sl]
                    sq = zf
                    for pv, tv in ((pxv, txv), (pyv, tyv), (pzv, tzv)):
                        d = jnp.abs(pv[sl] - tv[sl])
                        d = jnp.minimum(d, 1.0)
                        w = jnp.minimum(d, 1.0 - d)
                        sq = sq + w * w
                    si = lane_a + ids
                    plsc.addupdate_scatter(acc_a, [si], sq)
                    gmask = (base + o + iota) < N
                    plsc.addupdate_scatter(acc_a, [si + B], onesf, mask=gmask)
                return c2
            lax.fori_loop(0, _CH_A // (_L * _AUNROLL), astep, None)

        def reda(v, carry):
            o = v * _L
            t = [acc_a[pl.ds(l * ASTRIDE + o, _L)] for l in range(_L)]
            while len(t) > 1:
                t = [t[i] + t[i + 1] for i in range(0, len(t), 2)]
            red_a[pl.ds(o, _L)] = t[0]
            return carry
        lax.fori_loop(0, (2 * B) // _L, reda, None)

        pltpu.sync_copy(red_a, outa_h.at[wid])

    return sck(px, py, pz, tx, ty, tz, abatch)


def _sc_edges(pelx, pely, pelz, ebatch, E, B):
    KE = -(-E // (_NW * _CH_E))
    ESTRIDE = 3 * B + 1

    @functools.partial(
        pl.kernel,
        out_type=jax.ShapeDtypeStruct((_NW, 3 * B), jnp.float32),
        mesh=_sc_mesh(),
        compiler_params=pltpu.CompilerParams(needs_layout_passes=False),
        scratch_types=[
            pltpu.VMEM((_L * ESTRIDE,), jnp.float32),
            pltpu.VMEM((_CH_E,), jnp.float32),
            pltpu.VMEM((_CH_E,), jnp.float32),
            pltpu.VMEM((_CH_E,), jnp.float32),
            pltpu.VMEM((_CH_E,), jnp.int32),
            pltpu.VMEM((_CH_E,), jnp.float32),
            pltpu.VMEM((_CH_E,), jnp.float32),
            pltpu.VMEM((_CH_E,), jnp.float32),
            pltpu.VMEM((_CH_E,), jnp.int32),
            pltpu.VMEM((3 * B,), jnp.float32),
            pltpu.SemaphoreType.DMA,
            pltpu.SemaphoreType.DMA,
        ],
    )
    def sck(pelx_h, pely_h, pelz_h, ebatch_h, oute_h,
            acc_e,
            ex0, ey0, ez0, ei0, ex1, ey1, ez1, ei1,
            red_e, esem0, esem1):
        wid = lax.axis_index("s") * _NC + lax.axis_index("c")
        iota = lax.iota(jnp.int32, _L)
        zf = jnp.zeros((_L,), jnp.float32)
        lane_e = iota * ESTRIDE
        ebufs = ((ex0, ey0, ez0, ei0), (ex1, ey1, ez1, ei1))
        esems = (esem0, esem1)
        ebase0 = wid * (KE * _CH_E)

        def ecps(k, bufs):
            base = ebase0 + k * _CH_E
            return tuple(
                (src, src.at[pl.ds(base, _CH_E)], dst) for src, dst in
                zip((pelx_h, pely_h, pelz_h, ebatch_h), bufs))

        def fire(cps, sem):
            for _, sl, dst in cps:
                pltpu.async_copy(sl, dst, sem)

        def drain(cps, sem):
            for _, sl, dst in cps:
                pltpu.make_async_copy(sl, dst, sem).wait()

        fire(ecps(0, ebufs[0]), esems[0])

        def zero_e(i, carry):
            acc_e[pl.ds(i * _L, _L)] = zf
            return carry
        lax.fori_loop(0, _L * ESTRIDE // _L, zero_e, None)

        for k in range(KE):
            cb = k % 2
            nb = (k + 1) % 2
            if k + 1 < KE:
                fire(ecps(k + 1, ebufs[nb]), esems[nb])
            drain(ecps(k, ebufs[cb]), esems[cb])
            xv, yv, zv, iv = ebufs[cb]

            def estep(j, c2, xv=xv, yv=yv, zv=zv, iv=iv):
                for u in range(_EUNROLL):
                    o = (j * _EUNROLL + u) * _L
                    sl = pl.ds(o, _L)
                    ids = iv[sl]
                    si = lane_e + ids
                    plsc.addupdate_scatter(acc_e, [si], xv[sl])
                    plsc.addupdate_scatter(acc_e, [si + B], yv[sl])
                    plsc.addupdate_scatter(acc_e, [si + 2 * B], zv[sl])
                return c2
            lax.fori_loop(0, _CH_E // (_L * _EUNROLL), estep, None)

        def rede(v, carry):
            o = v * _L
            t = [acc_e[pl.ds(l * ESTRIDE + o, _L)] for l in range(_L)]
            while len(t) > 1:
                t = [t[i] + t[i + 1] for i in range(0, len(t), 2)]
            red_e[pl.ds(o, _L)] = t[0]
            return carry
        lax.fori_loop(0, (3 * B) // _L, rede, None)

        pltpu.sync_copy(red_e, oute_h.at[wid])

    return sck(pelx, pely, pelz, ebatch)


def _combine_body(oute, outa, lat, noise, out):
    B = noise.shape[1]
    e = oute[...]
    S = [jnp.sum(e[:, i * B:(i + 1) * B], axis=0) for i in range(3)]
    a = outa[...]
    seg_sum = jnp.sum(a[:, 0:B], axis=0)
    seg_cnt = jnp.sum(a[:, B:2 * B], axis=0)
    err_x = seg_sum / jnp.maximum(seg_cnt, 1.0)
    mean_err_x = jnp.sum(err_x) / B

    m = [[lat[3 * i + j, :] * S[i] for j in range(3)] for i in range(3)]
    A = [[None] * 3 for _ in range(3)]
    for j in range(3):
        for k in range(j, 3):
            A[j][k] = m[0][j] * m[0][k] + m[1][j] * m[1][k] + m[2][j] * m[2][k]
            A[k][j] = A[j][k]
    one = jnp.ones((B,), jnp.float32)
    zero = jnp.zeros((B,), jnp.float32)
    V = [[one if i == j else zero for j in range(3)] for i in range(3)]

    for _ in range(_NSWEEP):
        for (p, q) in ((0, 1), (0, 2), (1, 2)):
            app, aqq, apq = A[p][p], A[q][q], A[p][q]
            nz = apq != 0.0
            apq_s = jnp.where(nz, apq, 1.0)
            tau = (aqq - app) / (2.0 * apq_s)
            sgn = jnp.where(tau >= 0.0, 1.0, -1.0)
            t = sgn / (jnp.abs(tau) + jnp.sqrt(1.0 + tau * tau))
            t = jnp.where(nz, t, 0.0)
            c = 1.0 / jnp.sqrt(1.0 + t * t)
            s = t * c
            r = 3 - p - q
            arp = c * A[r][p] - s * A[r][q]
            arq = s * A[r][p] + c * A[r][q]
            A[p][p] = app - t * apq
            A[q][q] = aqq + t * apq
            A[p][q] = zero
            A[q][p] = zero
            A[r][p] = arp
            A[p][r] = arp
            A[r][q] = arq
            A[q][r] = arq
            for i in range(3):
                vip = c * V[i][p] - s * V[i][q]
                viq = s * V[i][p] + c * V[i][q]
                V[i][p] = vip
                V[i][q] = viq

    rt = [jnp.sqrt(jnp.maximum(A[k][k], 0.0)) for k in range(3)]

    def sym(i, j):
        return (V[i][0] * rt[0] * V[j][0]
                + V[i][1] * rt[1] * V[j][1]
                + V[i][2] * rt[2] * V[j][2])

    comps = (sym(0, 0), sym(1, 1), sym(2, 2), sym(0, 1), sym(0, 2), sym(1, 2))
    tot = jnp.zeros((), jnp.float32)
    for ci in range(6):
        d = comps[ci] - noise[ci, :]
        tot = tot + jnp.sum(d * d)
    err_l = tot / (6.0 * B)
    out[0, 0] = mean_err_x + err_l


def _tc_combine(oute, outa, lat_t, noise_t, interpret=False):
    res = pl.pallas_call(
        _combine_body,
        out_shape=jax.ShapeDtypeStruct((1, 1), jnp.float32),
        out_specs=pl.BlockSpec(memory_space=pltpu.SMEM),
        interpret=interpret,
    )(oute, outa, lat_t, noise_t)
    return res[0, 0]


def kernel(pred_frac_eps_x, target_frac_eps_x, atom_batch, neighbor_direction,
           pred_edge_distance_score, lattice, batch_of_edge,
           symmetric_vector_noise):
    B = lattice.shape[0]
    N = pred_frac_eps_x.shape[0]
    E = neighbor_direction.shape[0]
    KE = -(-E // (_NW * _CH_E))
    KA = -(-N // (_NW * _CH_A))
    EP = _NW * KE * _CH_E
    AP = _NW * KA * _CH_A
    pred = pred_frac_eps_x.astype(jnp.float32)
    targ = target_frac_eps_x.astype(jnp.float32)
    nd = neighbor_direction.astype(jnp.float32)
    sc = pred_edge_distance_score.astype(jnp.float32)[:, 0]
    ab = jnp.pad(atom_batch.astype(jnp.int32), (0, AP - N))
    eb = jnp.pad(batch_of_edge.astype(jnp.int32), (0, EP - E))
    px, py, pz, tx, ty, tz = (jnp.pad(a, (0, AP - N)) for a in (
        pred[:, 0], pred[:, 1], pred[:, 2],
        targ[:, 0], targ[:, 1], targ[:, 2]))
    outa = _sc_atoms(px, py, pz, tx, ty, tz, ab, N, B)
    pelx, pely, pelz = (jnp.pad(nd[:, i] * sc, (0, EP - E)) for i in range(3))
    oute = _sc_edges(pelx, pely, pelz, eb, E, B)
    lat_t = jnp.transpose(jnp.reshape(lattice.astype(jnp.float32), (B, 9)))
    noise_t = jnp.transpose(symmetric_vector_noise.astype(jnp.float32))
    return _tc_combine(oute, outa, lat_t, noise_t)
